# Initial kernel scaffold; baseline (speedup 1.0000x reference)
import jax
import jax.numpy as jnp
from jax.experimental import pallas as pl


def kernel(features, coords, condition, W1, b1, g1, be1, W2, b2, g2, be2):
    raise NotImplementedError("write your pallas kernel here")


# speedup vs baseline: 11.5138x; 11.5138x over previous
import functools

import jax
import jax.numpy as jnp
from jax import lax
from jax.experimental import pallas as pl
from jax.experimental.pallas import tpu as pltpu
from jax.experimental.pallas import tpu_sc as plsc

B_SZ, N_PTS, M_CTR, K_NBR = 4, 8192, 1024, 32
C_IN = 32
RADIUS = 0.2
C1_CH, C2_CH = 32, 64
N_GROUPS = 8
SUB, LANE = 64, 128
N_CHUNKS = N_PTS // LANE
CBLK = 128
D_PAD = 128
NC_SC, NS_SC = 2, 16
NW_SC = NC_SC * NS_SC
GCHUNK = 128


def _fps_body(xs_ref, ys_ref, zs_ref, ctr_ref):
    xs = xs_ref[...]
    ys = ys_ref[...]
    zs = zs_ref[...]
    row_i = lax.broadcasted_iota(jnp.int32, (1, SUB, LANE), 1)
    col_i = lax.broadcasted_iota(jnp.int32, (1, SUB, LANE), 2)
    flat_i = row_i * LANE + col_i
    m_iota = lax.broadcasted_iota(jnp.int32, (1, M_CTR), 1)

    def step(i, carry):
        dists, cur, cx, cy, cz = carry
        onehot = flat_i == cur
        lx = jnp.sum(jnp.where(onehot, xs, 0.0), axis=(1, 2), keepdims=True)
        ly = jnp.sum(jnp.where(onehot, ys, 0.0), axis=(1, 2), keepdims=True)
        lz = jnp.sum(jnp.where(onehot, zs, 0.0), axis=(1, 2), keepdims=True)
        sel = m_iota == i
        cx = jnp.where(sel, jnp.reshape(lx, (B_SZ, 1)), cx)
        cy = jnp.where(sel, jnp.reshape(ly, (B_SZ, 1)), cy)
        cz = jnp.where(sel, jnp.reshape(lz, (B_SZ, 1)), cz)
        dx = xs - lx
        dy = ys - ly
        dz = zs - lz
        d = dx * dx + (dy * dy + dz * dz)
        dists = jnp.minimum(dists, d)
        maxv = jnp.max(dists, axis=(1, 2), keepdims=True)
        nxt = jnp.min(
            jnp.where(dists == maxv, flat_i, N_PTS), axis=(1, 2), keepdims=True
        )
        return dists, nxt, cx, cy, cz

    init = (
        jnp.full((B_SZ, SUB, LANE), 1e10, jnp.float32),
        jnp.zeros((B_SZ, 1, 1), jnp.int32),
        jnp.zeros((B_SZ, M_CTR), jnp.float32),
        jnp.zeros((B_SZ, M_CTR), jnp.float32),
        jnp.zeros((B_SZ, M_CTR), jnp.float32),
    )
    _, _, cx, cy, cz = lax.fori_loop(0, M_CTR, step, init)
    ctr_ref[...] = jnp.concatenate(
        [cx[:, None, :], cy[:, None, :], cz[:, None, :]], axis=1
    )


def _run_fps(coords):
    xs = coords[:, 0, :].reshape(B_SZ, SUB, LANE)
    ys = coords[:, 1, :].reshape(B_SZ, SUB, LANE)
    zs = coords[:, 2, :].reshape(B_SZ, SUB, LANE)
    return pl.pallas_call(
        _fps_body,
        out_shape=jax.ShapeDtypeStruct((B_SZ, 3, M_CTR), jnp.float32),
    )(xs, ys, zs)


def _ballq_body(pts_ref, ctr_ref, idx_ref):
    ctr = ctr_ref[0]
    r2 = jnp.float32(RADIUS * RADIUS)

    c2 = jnp.transpose(jnp.sum(ctr * ctr, axis=0, keepdims=True))
    tri_r = lax.broadcasted_iota(jnp.int32, (LANE, LANE), 0)
    tri_c = lax.broadcasted_iota(jnp.int32, (LANE, LANE), 1)
    tri = (tri_r <= tri_c).astype(jnp.float32)
    k_iota = lax.broadcasted_iota(jnp.int32, (K_NBR, 1, 1), 0).astype(jnp.float32)
    lane_f = lax.broadcasted_iota(jnp.int32, (1, 1, LANE), 2).astype(jnp.float32)

    def chunk(state):
        j, cnt, acc = state
        ptsj = pts_ref[0, j]
        p2 = jnp.sum(ptsj * ptsj, axis=0, keepdims=True)
        cp = lax.dot_general(
            ctr, ptsj, (((0,), (0,)), ((), ())),
            preferred_element_type=jnp.float32,
        )
        d2 = (c2 + p2) - 2.0 * cp
        m = (d2 < r2).astype(jnp.float32)
        pre = lax.dot_general(
            m, tri, (((1,), (0,)), ((), ())),
            preferred_element_type=jnp.float32,
        )
        grank = (cnt + pre) - m
        valid = (m > 0.0) & (grank < jnp.float32(K_NBR))
        sel = (grank[None] == k_iota) & valid[None]
        nval = jnp.float32(j * LANE) + lane_f
        acc = acc + jnp.sum(jnp.where(sel, nval, 0.0), axis=2)
        cnt = cnt + pre[:, LANE - 1:LANE]
        return j + 1, cnt, acc

    def cond(state):
        j, cnt, _ = state
        return (j < N_CHUNKS) & (jnp.min(cnt) < jnp.float32(K_NBR))

    init = (
        jnp.int32(0),
        jnp.zeros((CBLK, 1), jnp.float32),
        jnp.zeros((K_NBR, CBLK), jnp.float32),
    )
    _, cnt, acc = lax.while_loop(cond, chunk, init)

    cnt_t = jnp.transpose(cnt)
    k_col = lax.broadcasted_iota(jnp.int32, (K_NBR, 1), 0).astype(jnp.float32)
    padded = jnp.where(k_col < cnt_t, acc, acc[0:1, :])
    idx_ref[0] = padded.astype(jnp.int32)


def _run_ballq(coords_chunked, centers_pad):
    return pl.pallas_call(
        _ballq_body,
        grid=(B_SZ, M_CTR // CBLK),
        in_specs=[
            pl.BlockSpec((1, N_CHUNKS, 8, LANE), lambda b, c: (b, 0, 0, 0)),
            pl.BlockSpec((1, 8, CBLK), lambda b, c: (b, 0, c)),
        ],
        out_specs=pl.BlockSpec((1, K_NBR, CBLK), lambda b, c: (b, 0, c)),
        out_shape=jax.ShapeDtypeStruct((B_SZ, K_NBR, M_CTR), jnp.int32),
    )(coords_chunked, centers_pad)


def _sc_gather_body(table_hbm, idx_hbm, out_hbm, idx_v, rows_v, sem):
    wid = lax.axis_index("s") * NC_SC + lax.axis_index("c")
    rows_per_w = (B_SZ * K_NBR * M_CTR) // NW_SC
    base = wid * rows_per_w

    def it(t, _):
        off = base + t * GCHUNK
        pltpu.sync_copy(idx_hbm.at[pl.ds(off, GCHUNK)], idx_v)
        pltpu.async_copy(table_hbm.at[idx_v], rows_v, sem).wait()
        pltpu.sync_copy(rows_v, out_hbm.at[pl.ds(off, GCHUNK)])
        return 0

    lax.fori_loop(0, rows_per_w // GCHUNK, it, 0)


def _run_sc_gather(table, gidx):
    mesh = plsc.VectorSubcoreMesh(core_axis_name="c", subcore_axis_name="s")
    fn = functools.partial(
        pl.kernel,
        mesh=mesh,
        out_type=jax.ShapeDtypeStruct((B_SZ * K_NBR * M_CTR, D_PAD),
                                      jnp.float32),
        scratch_types=[
            pltpu.VMEM((GCHUNK,), jnp.int32),
            pltpu.VMEM((GCHUNK, D_PAD), jnp.float32),
            pltpu.SemaphoreType.DMA,
        ],
    )(_sc_gather_body)
    return fn(table, gidx)


def _group_norm(h, n_ch, gamma, beta):
    per = n_ch // N_GROUPS
    parts = []
    for g in range(N_GROUPS):
        x = h[g * per:(g + 1) * per, :]
        mu = jnp.mean(x)
        xc = x - mu
        var = jnp.mean(xc * xc)
        parts.append(xc * lax.rsqrt(var + 1e-5))
    hn = jnp.concatenate(parts, axis=0)
    return jnp.maximum(hn * gamma + beta, 0.0)


def _mlp_body(g_ref, ctr_ref, w1_ref, b1_ref, g1_ref, be1_ref,
              w2_ref, b2_ref, g2_ref, be2_ref, out_ref):
    gth = g_ref[0]
    ctr = ctr_ref[0]
    w1 = w1_ref[...]
    h1 = lax.dot_general(
        w1, gth, (((1,), (1,)), ((), ())),
        preferred_element_type=jnp.float32,
    )
    ctr_w = lax.dot_general(
        w1[:, 0:3], ctr, (((1,), (0,)), ((), ())),
        preferred_element_type=jnp.float32,
    )
    ctr_tile = jnp.concatenate([ctr_w] * K_NBR, axis=1)
    h1 = (h1 - ctr_tile) + b1_ref[...]
    h1 = _group_norm(h1, C1_CH, g1_ref[...], be1_ref[...])

    h2 = lax.dot_general(
        w2_ref[...], h1, (((1,), (0,)), ((), ())),
        preferred_element_type=jnp.float32,
    ) + b2_ref[...]
    h2 = _group_norm(h2, C2_CH, g2_ref[...], be2_ref[...])

    out = h2[:, 0:M_CTR]
    for k in range(1, K_NBR):
        out = jnp.maximum(out, h2[:, k * M_CTR:(k + 1) * M_CTR])
    out_ref[0] = out


def _run_mlp(gathered, centers, w1p, b1, g1, be1, w2, b2, g2, be2):
    km = K_NBR * M_CTR
    return pl.pallas_call(
        _mlp_body,
        grid=(B_SZ,),
        in_specs=[
            pl.BlockSpec((1, km, D_PAD), lambda b: (b, 0, 0)),
            pl.BlockSpec((1, 3, M_CTR), lambda b: (b, 0, 0)),
            pl.BlockSpec((C1_CH, D_PAD), lambda b: (0, 0)),
            pl.BlockSpec((C1_CH, 1), lambda b: (0, 0)),
            pl.BlockSpec((C1_CH, 1), lambda b: (0, 0)),
            pl.BlockSpec((C1_CH, 1), lambda b: (0, 0)),
            pl.BlockSpec((C2_CH, C1_CH), lambda b: (0, 0)),
            pl.BlockSpec((C2_CH, 1), lambda b: (0, 0)),
            pl.BlockSpec((C2_CH, 1), lambda b: (0, 0)),
            pl.BlockSpec((C2_CH, 1), lambda b: (0, 0)),
        ],
        out_specs=pl.BlockSpec((1, C2_CH, M_CTR), lambda b: (b, 0, 0)),
        out_shape=jax.ShapeDtypeStruct((B_SZ, C2_CH, M_CTR), jnp.float32),
    )(gathered, centers, w1p, b1, g1, be1, w2, b2, g2, be2)


def kernel(features, coords, condition, W1, b1, g1, be1, W2, b2, g2, be2):
    centers = _run_fps(coords)

    coords_pad = jnp.concatenate(
        [coords, jnp.zeros((B_SZ, 5, N_PTS), jnp.float32)], axis=1
    )
    coords_chunked = jnp.transpose(
        coords_pad.reshape(B_SZ, 8, N_CHUNKS, LANE), (0, 2, 1, 3)
    )
    centers_pad = jnp.concatenate(
        [centers, jnp.zeros((B_SZ, 5, M_CTR), jnp.float32)], axis=1
    )
    idx = _run_ballq(coords_chunked, centers_pad)

    payload = jnp.concatenate(
        [
            jnp.transpose(coords, (0, 2, 1)),
            jnp.transpose(features, (0, 2, 1)),
            jnp.zeros((B_SZ, N_PTS, D_PAD - 3 - C_IN), jnp.float32),
        ],
        axis=-1,
    ).reshape(B_SZ * N_PTS, D_PAD)
    gidx = (idx + (jnp.arange(B_SZ, dtype=jnp.int32) * N_PTS)[:, None, None])
    gidx = gidx.reshape(B_SZ * K_NBR * M_CTR)
    gathered = _run_sc_gather(payload, gidx)
    gathered = gathered.reshape(B_SZ, K_NBR * M_CTR, D_PAD)

    w1p = jnp.concatenate(
        [W1, jnp.zeros((C1_CH, D_PAD - W1.shape[1]), jnp.float32)], axis=1
    )
    out = _run_mlp(
        gathered, centers, w1p,
        b1[:, None], g1[:, None], be1[:, None],
        W2, b2[:, None], g2[:, None], be2[:, None],
    )
    return (out, centers, condition)


# speedup vs baseline: 12.6148x; 1.0956x over previous
import functools

import jax
import jax.numpy as jnp
from jax import lax
from jax.experimental import pallas as pl
from jax.experimental.pallas import tpu as pltpu
from jax.experimental.pallas import tpu_sc as plsc

B_SZ, N_PTS, M_CTR, K_NBR = 4, 8192, 1024, 32
C_IN = 32
RADIUS = 0.2
C1_CH, C2_CH = 32, 64
N_GROUPS = 8
SUB, LANE = 64, 128
N_CHUNKS = N_PTS // LANE
CBLK = 128
D_PAD = 128
NC_SC, NS_SC = 2, 16
NW_SC = NC_SC * NS_SC
GCHUNK = 128


def _fps_body(xs_ref, ys_ref, zs_ref, ctr_ref):
    xs = xs_ref[...]
    ys = ys_ref[...]
    zs = zs_ref[...]
    row_i = lax.broadcasted_iota(jnp.int32, (1, SUB, LANE), 1)
    col_i = lax.broadcasted_iota(jnp.int32, (1, SUB, LANE), 2)
    flat_i = row_i * LANE + col_i
    m_iota = lax.broadcasted_iota(jnp.int32, (1, M_CTR), 1)

    def step(i, carry):
        dists, cur, cx, cy, cz = carry
        onehot = flat_i == cur
        lx = jnp.sum(jnp.where(onehot, xs, 0.0), axis=(1, 2), keepdims=True)
        ly = jnp.sum(jnp.where(onehot, ys, 0.0), axis=(1, 2), keepdims=True)
        lz = jnp.sum(jnp.where(onehot, zs, 0.0), axis=(1, 2), keepdims=True)
        sel = m_iota == i
        cx = jnp.where(sel, jnp.reshape(lx, (B_SZ, 1)), cx)
        cy = jnp.where(sel, jnp.reshape(ly, (B_SZ, 1)), cy)
        cz = jnp.where(sel, jnp.reshape(lz, (B_SZ, 1)), cz)
        dx = xs - lx
        dy = ys - ly
        dz = zs - lz
        d = dx * dx + (dy * dy + dz * dz)
        dists = jnp.minimum(dists, d)
        maxv = jnp.max(dists, axis=(1, 2), keepdims=True)
        nxt = jnp.min(
            jnp.where(dists == maxv, flat_i, N_PTS), axis=(1, 2), keepdims=True
        )
        return dists, nxt, cx, cy, cz

    init = (
        jnp.full((B_SZ, SUB, LANE), 1e10, jnp.float32),
        jnp.zeros((B_SZ, 1, 1), jnp.int32),
        jnp.zeros((B_SZ, M_CTR), jnp.float32),
        jnp.zeros((B_SZ, M_CTR), jnp.float32),
        jnp.zeros((B_SZ, M_CTR), jnp.float32),
    )
    _, _, cx, cy, cz = lax.fori_loop(0, M_CTR, step, init)
    ctr_ref[...] = jnp.concatenate(
        [cx[:, None, :], cy[:, None, :], cz[:, None, :]], axis=1
    )


def _run_fps(coords):
    xs = coords[:, 0, :].reshape(B_SZ, SUB, LANE)
    ys = coords[:, 1, :].reshape(B_SZ, SUB, LANE)
    zs = coords[:, 2, :].reshape(B_SZ, SUB, LANE)
    return pl.pallas_call(
        _fps_body,
        out_shape=jax.ShapeDtypeStruct((B_SZ, 3, M_CTR), jnp.float32),
    )(xs, ys, zs)


def _ballq_body(pts_ref, ctr_ref, idx_ref):
    ctr = ctr_ref[0]
    r2 = jnp.float32(RADIUS * RADIUS)

    c2 = jnp.transpose(jnp.sum(ctr * ctr, axis=0, keepdims=True))
    tri_r = lax.broadcasted_iota(jnp.int32, (LANE, LANE), 0)
    tri_c = lax.broadcasted_iota(jnp.int32, (LANE, LANE), 1)
    tri = (tri_r <= tri_c).astype(jnp.float32)
    lane_f = lax.broadcasted_iota(jnp.int32, (1, LANE), 1).astype(jnp.float32)

    def chunk(state):
        j, cnt, acc = state
        ptsj = pts_ref[0, j]
        p2 = jnp.sum(ptsj * ptsj, axis=0, keepdims=True)
        cp = lax.dot_general(
            ctr, ptsj, (((0,), (0,)), ((), ())),
            preferred_element_type=jnp.float32,
        )
        d2 = (c2 + p2) - 2.0 * cp
        m = (d2 < r2).astype(jnp.float32)
        pre = lax.dot_general(
            m, tri, (((1,), (0,)), ((), ())),
            preferred_element_type=jnp.float32,
        )
        grank = (cnt + pre) - m
        valid = (m > 0.0) & (grank < jnp.float32(K_NBR))
        nval = jnp.float32(j * LANE) + lane_f
        cols = []
        for k in range(K_NBR):
            selk = valid & (grank == jnp.float32(k))
            cols.append(jnp.sum(jnp.where(selk, nval, 0.0), axis=1,
                                keepdims=True))
        acc = acc + jnp.concatenate(cols, axis=1)
        cnt = cnt + pre[:, LANE - 1:LANE]
        return j + 1, cnt, acc

    def cond(state):
        j, cnt, _ = state
        return (j < N_CHUNKS) & (jnp.min(cnt) < jnp.float32(K_NBR))

    init = (
        jnp.int32(0),
        jnp.zeros((CBLK, 1), jnp.float32),
        jnp.zeros((CBLK, K_NBR), jnp.float32),
    )
    _, cnt, acc = lax.while_loop(cond, chunk, init)

    k_row = lax.broadcasted_iota(jnp.int32, (1, K_NBR), 1).astype(jnp.float32)
    padded = jnp.where(k_row < cnt, acc, acc[:, 0:1])
    idx_ref[0] = padded.astype(jnp.int32)


def _run_ballq(coords_chunked, centers_pad):
    return pl.pallas_call(
        _ballq_body,
        grid=(B_SZ, M_CTR // CBLK),
        in_specs=[
            pl.BlockSpec((1, N_CHUNKS, 8, LANE), lambda b, c: (b, 0, 0, 0)),
            pl.BlockSpec((1, 8, CBLK), lambda b, c: (b, 0, c)),
        ],
        out_specs=pl.BlockSpec((1, CBLK, K_NBR), lambda b, c: (b, c, 0)),
        out_shape=jax.ShapeDtypeStruct((B_SZ, M_CTR, K_NBR), jnp.int32),
    )(coords_chunked, centers_pad)


def _sc_gather_body(table_hbm, idx_hbm, out_hbm, idx_v, rows_v, sem):
    wid = lax.axis_index("s") * NC_SC + lax.axis_index("c")
    rows_per_w = (B_SZ * K_NBR * M_CTR) // NW_SC
    base = wid * rows_per_w

    def it(t, _):
        off = base + t * GCHUNK
        pltpu.sync_copy(idx_hbm.at[pl.ds(off, GCHUNK)], idx_v)
        pltpu.async_copy(table_hbm.at[idx_v], rows_v, sem).wait()
        pltpu.sync_copy(rows_v, out_hbm.at[pl.ds(off, GCHUNK)])
        return 0

    lax.fori_loop(0, rows_per_w // GCHUNK, it, 0)


def _run_sc_gather(table, gidx):
    mesh = plsc.VectorSubcoreMesh(core_axis_name="c", subcore_axis_name="s")
    fn = functools.partial(
        pl.kernel,
        mesh=mesh,
        out_type=jax.ShapeDtypeStruct((B_SZ * K_NBR * M_CTR, D_PAD),
                                      jnp.float32),
        scratch_types=[
            pltpu.VMEM((GCHUNK,), jnp.int32),
            pltpu.VMEM((GCHUNK, D_PAD), jnp.float32),
            pltpu.SemaphoreType.DMA,
        ],
    )(_sc_gather_body)
    return fn(table, gidx)


def _group_norm(h, n_ch, gamma, beta):
    per = n_ch // N_GROUPS
    parts = []
    for g in range(N_GROUPS):
        x = h[g * per:(g + 1) * per, :]
        mu = jnp.mean(x)
        xc = x - mu
        var = jnp.mean(xc * xc)
        parts.append(xc * lax.rsqrt(var + 1e-5))
    hn = jnp.concatenate(parts, axis=0)
    return jnp.maximum(hn * gamma + beta, 0.0)


def _mlp_body(g_ref, ctr_ref, w1_ref, b1_ref, g1_ref, be1_ref,
              w2_ref, b2_ref, g2_ref, be2_ref, out_ref):
    gth = g_ref[0]
    ctr = ctr_ref[0]
    w1 = w1_ref[...]
    h1 = lax.dot_general(
        w1, gth, (((1,), (1,)), ((), ())),
        preferred_element_type=jnp.float32,
    )
    ctr_w = lax.dot_general(
        w1[:, 0:3], ctr, (((1,), (0,)), ((), ())),
        preferred_element_type=jnp.float32,
    )
    ctr_tile = jnp.concatenate([ctr_w] * K_NBR, axis=1)
    h1 = (h1 - ctr_tile) + b1_ref[...]
    h1 = _group_norm(h1, C1_CH, g1_ref[...], be1_ref[...])

    h2 = lax.dot_general(
        w2_ref[...], h1, (((1,), (0,)), ((), ())),
        preferred_element_type=jnp.float32,
    ) + b2_ref[...]
    h2 = _group_norm(h2, C2_CH, g2_ref[...], be2_ref[...])

    out = h2[:, 0:M_CTR]
    for k in range(1, K_NBR):
        out = jnp.maximum(out, h2[:, k * M_CTR:(k + 1) * M_CTR])
    out_ref[0] = out


def _run_mlp(gathered, centers, w1p, b1, g1, be1, w2, b2, g2, be2):
    km = K_NBR * M_CTR
    return pl.pallas_call(
        _mlp_body,
        grid=(B_SZ,),
        in_specs=[
            pl.BlockSpec((1, km, D_PAD), lambda b: (b, 0, 0)),
            pl.BlockSpec((1, 3, M_CTR), lambda b: (b, 0, 0)),
            pl.BlockSpec((C1_CH, D_PAD), lambda b: (0, 0)),
            pl.BlockSpec((C1_CH, 1), lambda b: (0, 0)),
            pl.BlockSpec((C1_CH, 1), lambda b: (0, 0)),
            pl.BlockSpec((C1_CH, 1), lambda b: (0, 0)),
            pl.BlockSpec((C2_CH, C1_CH), lambda b: (0, 0)),
            pl.BlockSpec((C2_CH, 1), lambda b: (0, 0)),
            pl.BlockSpec((C2_CH, 1), lambda b: (0, 0)),
            pl.BlockSpec((C2_CH, 1), lambda b: (0, 0)),
        ],
        out_specs=pl.BlockSpec((1, C2_CH, M_CTR), lambda b: (b, 0, 0)),
        out_shape=jax.ShapeDtypeStruct((B_SZ, C2_CH, M_CTR), jnp.float32),
    )(gathered, centers, w1p, b1, g1, be1, w2, b2, g2, be2)


def kernel(features, coords, condition, W1, b1, g1, be1, W2, b2, g2, be2):
    centers = _run_fps(coords)

    coords_pad = jnp.concatenate(
        [coords, jnp.zeros((B_SZ, 5, N_PTS), jnp.float32)], axis=1
    )
    coords_chunked = jnp.transpose(
        coords_pad.reshape(B_SZ, 8, N_CHUNKS, LANE), (0, 2, 1, 3)
    )
    centers_pad = jnp.concatenate(
        [centers, jnp.zeros((B_SZ, 5, M_CTR), jnp.float32)], axis=1
    )
    idx = jnp.transpose(
        _run_ballq(coords_chunked, centers_pad), (0, 2, 1)
    )

    payload = jnp.concatenate(
        [
            jnp.transpose(coords, (0, 2, 1)),
            jnp.transpose(features, (0, 2, 1)),
            jnp.zeros((B_SZ, N_PTS, D_PAD - 3 - C_IN), jnp.float32),
        ],
        axis=-1,
    ).reshape(B_SZ * N_PTS, D_PAD)
    gidx = (idx + (jnp.arange(B_SZ, dtype=jnp.int32) * N_PTS)[:, None, None])
    gidx = gidx.reshape(B_SZ * K_NBR * M_CTR)
    gathered = _run_sc_gather(payload, gidx)
    gathered = gathered.reshape(B_SZ, K_NBR * M_CTR, D_PAD)

    w1p = jnp.concatenate(
        [W1, jnp.zeros((C1_CH, D_PAD - W1.shape[1]), jnp.float32)], axis=1
    )
    out = _run_mlp(
        gathered, centers, w1p,
        b1[:, None], g1[:, None], be1[:, None],
        W2, b2[:, None], g2[:, None], be2[:, None],
    )
    return (out, centers, condition)
